# baseline (speedup 1.0000x reference)
import functools

import jax
import jax.numpy as jnp
from jax import lax
from jax.experimental import pallas as pl
from jax.experimental.pallas import tpu as pltpu
from jax.experimental.pallas import tpu_sc as plsc

N_NODES = 10000
N_PAD = 10240
N_EDGES = 320000
D = 128

NC = 2
NS = 16
NW = NC * NS
CH = 80
EPT = N_EDGES // NW
CHUNKS = EPT // CH
RPS = N_PAD // NS

_MESH = plsc.VectorSubcoreMesh(
    core_axis_name="c", subcore_axis_name="s", num_cores=NC, num_subcores=NS
)


def _sc_accumulate(x_hbm, src_hbm, dst_hbm, out_hbm,
                   acc, src_v, dst_v, rows_v, sem, sem_s):
    c = lax.axis_index("c")
    s = lax.axis_index("s")
    w = c * NS + s

    pltpu.async_copy(src_hbm.at[pl.ds(w * EPT, EPT)], src_v, sem)
    pltpu.async_copy(dst_hbm.at[w], dst_v, sem_s)

    def _zrow(r, carry):
        for b in range(2):
            for j in range(D // 16):
                rows_v[b, r, pl.ds(j * 16, 16)] = jnp.zeros((16,), jnp.float32)
        return carry
    lax.fori_loop(0, CH, _zrow, 0)
    for k in range(RPS // CH):
        pltpu.sync_copy(rows_v.at[k % 2], acc.at[pl.ds(s * RPS + k * CH, CH)])
    plsc.subcore_barrier()

    pltpu.make_async_copy(src_hbm.at[pl.ds(w * EPT, EPT)], src_v, sem).wait()
    pltpu.make_async_copy(dst_hbm.at[w], dst_v, sem_s).wait()

    def _gather(i, buf):
        pltpu.async_copy(
            x_hbm.at[src_v.at[pl.ds(i * CH, CH)]], rows_v.at[buf], sem)

    def _gather_wait(i, buf):
        pltpu.make_async_copy(
            x_hbm.at[src_v.at[pl.ds(i * CH, CH)]], rows_v.at[buf], sem).wait()

    def _scatter(i, buf):
        pltpu.async_copy(rows_v.at[buf], acc.at[dst_v.at[i]], sem_s, add=True)

    def _scatter_wait(i, buf):
        pltpu.make_async_copy(
            rows_v.at[buf], acc.at[dst_v.at[i]], sem_s).wait()

    _gather(0, 0)

    def _chunk(i, carry):
        p = i % 2
        @pl.when(i >= 1)
        def _():
            _scatter_wait(i - 1, 1 - p)
        @pl.when(i + 1 < CHUNKS)
        def _():
            _gather(i + 1, 1 - p)
        _gather_wait(i, p)
        _scatter(i, p)
        return carry
    lax.fori_loop(0, CHUNKS, _chunk, 0)
    _scatter_wait(CHUNKS - 1, (CHUNKS - 1) % 2)

    plsc.subcore_barrier()
    pltpu.sync_copy(acc.at[pl.ds(s * RPS, RPS)],
                    out_hbm.at[c, pl.ds(s * RPS, RPS)])


_sc_kernel = functools.partial(
    pl.kernel,
    out_type=jax.ShapeDtypeStruct((NC, N_PAD, D), jnp.float32),
    mesh=_MESH,
    scratch_types=[
        pltpu.VMEM_SHARED((N_PAD, D), jnp.float32),
        pltpu.VMEM((EPT,), jnp.int32),
        pltpu.VMEM((CHUNKS, CH), jnp.int32),
        pltpu.VMEM((2, CH, D), jnp.float32),
        pltpu.SemaphoreType.DMA,
        pltpu.SemaphoreType.DMA,
    ],
)(_sc_accumulate)


def _tc_finalize(p_ref, k_ref, b_ref, o_ref):
    a = p_ref[0] + p_ref[1]
    y = jnp.dot(a, k_ref[...], preferred_element_type=jnp.float32)
    o_ref[...] = jnp.maximum(y + b_ref[...], 0.0)


def kernel(x, edge_index, kernel, bias):
    src = edge_index[0]
    dst = edge_index[1].reshape(NW, CHUNKS, CH)
    partials = _sc_kernel(x, src, dst)

    rows_blk = 2000
    grid = (N_NODES // rows_blk,)
    out = pl.pallas_call(
        _tc_finalize,
        grid=grid,
        in_specs=[
            pl.BlockSpec((NC, rows_blk, D), lambda i: (0, i, 0)),
            pl.BlockSpec((D, D), lambda i: (0, 0)),
            pl.BlockSpec((1, D), lambda i: (0, 0)),
        ],
        out_specs=pl.BlockSpec((rows_blk, D), lambda i: (i, 0)),
        out_shape=jax.ShapeDtypeStruct((N_NODES, D), jnp.float32),
    )(partials, kernel, bias.reshape(1, D))
    return out
